# baseline (speedup 1.0000x reference)
import functools

import jax
import jax.numpy as jnp
from jax import lax
from jax.experimental import pallas as pl
from jax.experimental.pallas import tpu as pltpu
from jax.experimental.pallas import tpu_sc as plsc

B = 16384
S = 50
D = 32
TOTAL = B * S

NC = 2
NS = 16
NW = NC * NS

IPG = 128
NBLK = TOTAL // IPG
PER_W = NBLK // NW
GPC = 8
N_CHUNKS = PER_W // GPC
CHUNK = GPC * IPG
BT = B // IPG


def _tanh16(x):
    e = jnp.exp(x + x)
    return 1.0 - 2.0 / (e + 1.0)


@functools.partial(
    pl.kernel,
    out_type=jax.ShapeDtypeStruct((S, D // 8, BT, 8, IPG), jnp.float32),
    mesh=plsc.VectorSubcoreMesh(core_axis_name="c", subcore_axis_name="s"),
    scratch_types=[
        pltpu.VMEM((GPC, IPG), jnp.int32),
        pltpu.VMEM((CHUNK, D), jnp.float32),
        pltpu.VMEM((D // 8, GPC, 8, IPG), jnp.float32),
        pltpu.SemaphoreType.DMA,
    ],
    compiler_params=pltpu.CompilerParams(use_tc_tiling_on_sc=False, needs_layout_passes=False),
)
def _gather_tanh(idx_hbm, tab_hbm, out_hbm, idx_v, rows_v, obuf, gsem):
    c = lax.axis_index("c")
    s_ax = lax.axis_index("s")
    wid = s_ax * NC + c
    r0_w = wid * PER_W

    def chunk_body(g, carry):
        r0 = r0_w + g * GPC
        sq = r0 // BT
        bt0 = r0 % BT
        pltpu.sync_copy(idx_hbm.at[pl.ds(r0, GPC)], idx_v)
        copies = []
        for j in range(GPC):
            copies.append(
                pltpu.async_copy(
                    tab_hbm.at[idx_v.at[j]],
                    rows_v.at[pl.ds(j * IPG, IPG)],
                    gsem,
                )
            )
        for cp in copies:
            cp.wait()

        lane = lax.iota(jnp.int32, 16)

        def tanh_t_body(d, carry2):
            dt = d >> 3
            ds = d & 7
            dvec = jnp.broadcast_to(d, (16,))
            for j in range(GPC):
                for u in range(IPG // 16):
                    ridx = lane + (j * IPG + u * 16)
                    x = plsc.load_gather(rows_v, [ridx, dvec])
                    obuf[dt, j, ds, pl.ds(u * 16, 16)] = _tanh16(x)
            return carry2

        lax.fori_loop(0, D, tanh_t_body, 0)
        for dt in range(D // 8):
            pltpu.sync_copy(obuf.at[dt], out_hbm.at[sq, dt, pl.ds(bt0, GPC)])
        return carry

    lax.fori_loop(0, N_CHUNKS, chunk_body, 0)


def kernel(question, word2vec):
    idx = jnp.transpose(question).reshape(NBLK, IPG).astype(jnp.int32)
    x = _gather_tanh(idx, word2vec)
    return x.transpose(0, 2, 4, 1, 3).reshape(S, B, D)


# speedup vs baseline: 1.2653x; 1.2653x over previous
import functools

import jax
import jax.numpy as jnp
from jax import lax
from jax.experimental import pallas as pl
from jax.experimental.pallas import tpu as pltpu
from jax.experimental.pallas import tpu_sc as plsc

B = 16384
S = 50
D = 32
TOTAL = B * S

NC = 2
NS = 16
NW = NC * NS

IPG = 128
NBLK = TOTAL // IPG
PER_W = NBLK // NW
GPC = 8
N_CHUNKS = PER_W // GPC
CHUNK = GPC * IPG
BT = B // IPG
OPAD = CHUNK + 1
RPI = 4


def _tanh16(x):
    e = jnp.exp(x + x)
    return 1.0 - 2.0 / (e + 1.0)


@functools.partial(
    pl.kernel,
    out_type=jax.ShapeDtypeStruct((S, D // 8, BT, 8, IPG), jnp.float32),
    mesh=plsc.VectorSubcoreMesh(core_axis_name="c", subcore_axis_name="s"),
    scratch_types=[
        pltpu.VMEM((GPC, IPG), jnp.int32),
        pltpu.VMEM((CHUNK, D), jnp.float32),
        pltpu.VMEM((D, OPAD), jnp.float32),
        pltpu.SemaphoreType.DMA,
        pltpu.SemaphoreType.DMA,
    ],
    compiler_params=pltpu.CompilerParams(
        use_tc_tiling_on_sc=False, needs_layout_passes=False
    ),
)
def _gather_tanh(idx_hbm, tab_hbm, out_hbm, idx_v, rows_v, obuf, gsem, osem):
    c = lax.axis_index("c")
    s_ax = lax.axis_index("s")
    wid = s_ax * NC + c
    r0_w = wid * PER_W

    def chunk_body(g, carry):
        r0 = r0_w + g * GPC
        sq = r0 // BT
        bt0 = r0 % BT
        pltpu.sync_copy(idx_hbm.at[pl.ds(r0, GPC)], idx_v)
        copies = []
        for j in range(GPC):
            copies.append(
                pltpu.async_copy(
                    tab_hbm.at[idx_v.at[j]],
                    rows_v.at[pl.ds(j * IPG, IPG)],
                    gsem,
                )
            )
        for cp in copies:
            cp.wait()

        lane = lax.iota(jnp.int32, 16)

        def tanh_t_body(i, carry2):
            r = i * RPI
            for u in range(RPI):
                rvec = jnp.broadcast_to(r + u, (16,))
                for h in range(D // 16):
                    x = rows_v[r + u, pl.ds(h * 16, 16)]
                    plsc.store_scatter(
                        obuf, [lane + h * 16, rvec], _tanh16(x)
                    )
            return carry2

        lax.fori_loop(0, CHUNK // RPI, tanh_t_body, 0)

        ocopies = []
        for dt in range(D // 8):
            for j in range(GPC):
                ocopies.append(
                    pltpu.async_copy(
                        obuf.at[pl.ds(dt * 8, 8), pl.ds(j * IPG, IPG)],
                        out_hbm.at[sq, dt, bt0 + j],
                        osem,
                    )
                )
        for cp in ocopies:
            cp.wait()
        return carry

    lax.fori_loop(0, N_CHUNKS, chunk_body, 0)


def kernel(question, word2vec):
    idx = jnp.transpose(question).reshape(NBLK, IPG).astype(jnp.int32)
    x = _gather_tanh(idx, word2vec)
    return x.transpose(0, 2, 4, 1, 3).reshape(S, B, D)


# speedup vs baseline: 1.2664x; 1.0009x over previous
import functools

import jax
import jax.numpy as jnp
from jax import lax
from jax.experimental import pallas as pl
from jax.experimental.pallas import tpu as pltpu
from jax.experimental.pallas import tpu_sc as plsc

B = 16384
S = 50
D = 32
TOTAL = B * S

NC = 2
NS = 16
NW = NC * NS

IPG = 128
NBLK = TOTAL // IPG
PER_W = NBLK // NW
GPC = 8
N_CHUNKS = PER_W // GPC
CHUNK = GPC * IPG
BT = B // IPG
OPAD = CHUNK + 1
RPI = 4


def _tanh16(x):
    e = jnp.exp(x + x)
    return 1.0 - 2.0 / (e + 1.0)


@functools.partial(
    pl.kernel,
    out_type=jax.ShapeDtypeStruct((S, D // 8, BT, 8 * IPG), jnp.float32),
    mesh=plsc.VectorSubcoreMesh(core_axis_name="c", subcore_axis_name="s"),
    scratch_types=[
        pltpu.VMEM((GPC, IPG), jnp.int32),
        pltpu.VMEM((CHUNK, D), jnp.float32),
        pltpu.VMEM((D, OPAD), jnp.float32),
        pltpu.SemaphoreType.DMA,
        pltpu.SemaphoreType.DMA,
    ],
    compiler_params=pltpu.CompilerParams(
        use_tc_tiling_on_sc=False, needs_layout_passes=False
    ),
)
def _gather_tanh(idx_hbm, tab_hbm, out_hbm, idx_v, rows_v, obuf, gsem, osem):
    c = lax.axis_index("c")
    s_ax = lax.axis_index("s")
    wid = s_ax * NC + c
    r0_w = wid * PER_W

    def chunk_body(g, carry):
        r0 = r0_w + g * GPC
        sq = r0 // BT
        bt0 = r0 % BT
        pltpu.sync_copy(idx_hbm.at[pl.ds(r0, GPC)], idx_v)
        copies = []
        for j in range(GPC):
            copies.append(
                pltpu.async_copy(
                    tab_hbm.at[idx_v.at[j]],
                    rows_v.at[pl.ds(j * IPG, IPG)],
                    gsem,
                )
            )
        for cp in copies:
            cp.wait()

        lane = lax.iota(jnp.int32, 16)

        def tanh_t_body(i, carry2):
            r = i * RPI
            for u in range(RPI):
                rvec = jnp.broadcast_to(r + u, (16,))
                for h in range(D // 16):
                    x = rows_v[r + u, pl.ds(h * 16, 16)]
                    plsc.store_scatter(
                        obuf, [lane + h * 16, rvec], _tanh16(x)
                    )
            return carry2

        lax.fori_loop(0, CHUNK // RPI, tanh_t_body, 0)

        ocopies = []
        for dt in range(D // 8):
            ocopies.append(
                pltpu.async_copy(
                    obuf.at[pl.ds(dt * 8, 8), pl.ds(0, CHUNK)],
                    out_hbm.at[sq, dt, pl.ds(bt0, GPC)],
                    osem,
                )
            )
        for cp in ocopies:
            cp.wait()
        return carry

    lax.fori_loop(0, N_CHUNKS, chunk_body, 0)


def kernel(question, word2vec):
    idx = jnp.transpose(question).reshape(NBLK, IPG).astype(jnp.int32)
    x = _gather_tanh(idx, word2vec)
    x = x.reshape(S, D // 8, BT, 8, IPG)
    return x.transpose(0, 2, 4, 1, 3).reshape(S, B, D)


# speedup vs baseline: 2.3916x; 1.8885x over previous
import functools

import jax
import jax.numpy as jnp
from jax import lax
from jax.experimental import pallas as pl
from jax.experimental.pallas import tpu as pltpu
from jax.experimental.pallas import tpu_sc as plsc

B = 16384
S = 50
D = 32
QV = 1000000
TOTAL = B * S

NC = 2
NS = 16
NW = NC * NS

IPG = 128
NBLK = TOTAL // IPG
PER_W = NBLK // NW
GPC = 20
N_CHUNKS = PER_W // GPC
CHUNK = GPC * IPG

VB = 4096
PGRID = (QV + VB - 1) // VB


def _pre_body(x_ref, o_ref):
    t = jnp.tanh(x_ref[...]).T
    t2 = t.reshape(VB // 4, 4, D)
    o_ref[...] = jnp.concatenate([t2[:, i, :] for i in range(4)], axis=1)


_pre = pl.pallas_call(
    _pre_body,
    grid=(PGRID,),
    in_specs=[pl.BlockSpec((D, VB), lambda i: (0, i))],
    out_specs=pl.BlockSpec((VB * D // IPG, IPG), lambda i: (i, 0)),
    out_shape=jax.ShapeDtypeStruct((QV * D // IPG, IPG), jnp.float32),
)


@functools.partial(
    pl.kernel,
    out_type=jax.ShapeDtypeStruct((TOTAL, D), jnp.float32),
    mesh=plsc.VectorSubcoreMesh(core_axis_name="c", subcore_axis_name="s"),
    scratch_types=[
        pltpu.VMEM((GPC, IPG), jnp.int32),
        pltpu.VMEM((CHUNK, D), jnp.float32),
        pltpu.SemaphoreType.DMA,
    ],
    compiler_params=pltpu.CompilerParams(use_tc_tiling_on_sc=False),
)
def _gather_rows(idx_hbm, tab_hbm, out_hbm, idx_v, rows_v, gsem):
    c = lax.axis_index("c")
    s_ax = lax.axis_index("s")
    wid = s_ax * NC + c
    r0_w = wid * PER_W

    def chunk_body(g, carry):
        r0 = r0_w + g * GPC
        pltpu.sync_copy(idx_hbm.at[pl.ds(r0, GPC)], idx_v)
        copies = []
        for j in range(GPC):
            copies.append(
                pltpu.async_copy(
                    tab_hbm.at[idx_v.at[j]],
                    rows_v.at[pl.ds(j * IPG, IPG)],
                    gsem,
                )
            )
        for cp in copies:
            cp.wait()
        pltpu.sync_copy(rows_v, out_hbm.at[pl.ds(r0 * IPG, CHUNK)])
        return carry

    lax.fori_loop(0, N_CHUNKS, chunk_body, 0)


def kernel(question, word2vec):
    idx = jnp.transpose(question).reshape(NBLK, IPG).astype(jnp.int32)
    tab = _pre(jnp.transpose(word2vec)).reshape(QV, D)
    out = _gather_rows(idx, tab)
    return out.reshape(S, B, D)


# speedup vs baseline: 2.4068x; 1.0063x over previous
import functools

import jax
import jax.numpy as jnp
from jax import lax
from jax.experimental import pallas as pl
from jax.experimental.pallas import tpu as pltpu
from jax.experimental.pallas import tpu_sc as plsc

B = 16384
S = 50
D = 32
QV = 1000000
TOTAL = B * S

NC = 2
NS = 16
NW = NC * NS

IPG = 128
NBLK = TOTAL // IPG
PER_W = NBLK // NW
GPC = 10
N_CHUNKS = PER_W // GPC
CHUNK = GPC * IPG

VB = 4096
PGRID = (QV + VB - 1) // VB


def _pre_body(x_ref, o_ref):
    t = jnp.tanh(x_ref[...]).T
    t2 = t.reshape(VB // 4, 4, D)
    o_ref[...] = jnp.concatenate([t2[:, i, :] for i in range(4)], axis=1)


_pre = pl.pallas_call(
    _pre_body,
    grid=(PGRID,),
    in_specs=[pl.BlockSpec((D, VB), lambda i: (0, i))],
    out_specs=pl.BlockSpec((VB * D // IPG, IPG), lambda i: (i, 0)),
    out_shape=jax.ShapeDtypeStruct((QV * D // IPG, IPG), jnp.float32),
)


@functools.partial(
    pl.kernel,
    out_type=jax.ShapeDtypeStruct((TOTAL, D), jnp.float32),
    mesh=plsc.VectorSubcoreMesh(core_axis_name="c", subcore_axis_name="s"),
    scratch_types=[
        pltpu.VMEM((GPC, IPG), jnp.int32),
        pltpu.VMEM((GPC, IPG), jnp.int32),
        pltpu.VMEM((CHUNK, D), jnp.float32),
        pltpu.VMEM((CHUNK, D), jnp.float32),
        pltpu.SemaphoreType.DMA,
        pltpu.SemaphoreType.DMA,
        pltpu.SemaphoreType.DMA,
        pltpu.SemaphoreType.DMA,
    ],
    compiler_params=pltpu.CompilerParams(use_tc_tiling_on_sc=False),
)
def _gather_rows(idx_hbm, tab_hbm, out_hbm, idx_a, idx_b, rows_a, rows_b,
                 gsem_a, gsem_b, osem_a, osem_b):
    c = lax.axis_index("c")
    s_ax = lax.axis_index("s")
    wid = s_ax * NC + c
    r0_w = wid * PER_W

    def fire(g, idx_v, rows_v, gsem):
        r0 = r0_w + g * GPC
        pltpu.sync_copy(idx_hbm.at[pl.ds(r0, GPC)], idx_v)
        for j in range(GPC):
            pltpu.async_copy(
                tab_hbm.at[idx_v.at[j]],
                rows_v.at[pl.ds(j * IPG, IPG)],
                gsem,
            )
        return r0

    def drain_gathers(rows_v, gsem):
        for j in range(GPC):
            pltpu.make_async_copy(
                tab_hbm.at[idx_a.at[0]],
                rows_v.at[pl.ds(j * IPG, IPG)],
                gsem,
            ).wait()

    def store(r0, rows_v, osem):
        pltpu.async_copy(rows_v, out_hbm.at[pl.ds(r0 * IPG, CHUNK)], osem)

    def drain_store(rows_v, osem):
        pltpu.make_async_copy(
            rows_v, out_hbm.at[pl.ds(0, CHUNK)], osem
        ).wait()

    def pair_body(i, carry):
        @pl.when(i > 0)
        def _():
            drain_store(rows_a, osem_a)

        ra = fire(2 * i, idx_a, rows_a, gsem_a)

        @pl.when(i > 0)
        def _():
            drain_store(rows_b, osem_b)

        rb = fire(2 * i + 1, idx_b, rows_b, gsem_b)
        drain_gathers(rows_a, gsem_a)
        store(ra, rows_a, osem_a)
        drain_gathers(rows_b, gsem_b)
        store(rb, rows_b, osem_b)
        return carry

    lax.fori_loop(0, N_CHUNKS // 2, pair_body, 0)
    drain_store(rows_a, osem_a)
    drain_store(rows_b, osem_b)


def kernel(question, word2vec):
    idx = jnp.transpose(question).reshape(NBLK, IPG).astype(jnp.int32)
    tab = _pre(jnp.transpose(word2vec)).reshape(QV, D)
    out = _gather_rows(idx, tab)
    return out.reshape(S, B, D)


# speedup vs baseline: 2.4280x; 1.0088x over previous
import functools

import jax
import jax.numpy as jnp
from jax import lax
from jax.experimental import pallas as pl
from jax.experimental.pallas import tpu as pltpu
from jax.experimental.pallas import tpu_sc as plsc

B = 16384
S = 50
D = 32
QV = 1000000
TOTAL = B * S

NC = 2
NS = 16
NW = NC * NS

IPG = 128
NBLK = TOTAL // IPG
PER_W = NBLK // NW
GPC = 10
N_CHUNKS = PER_W // GPC
CHUNK = GPC * IPG

VB = 16384
PGRID = (QV + VB - 1) // VB


def _pre_body(x_ref, o_ref):
    t = jnp.tanh(x_ref[...]).T
    t2 = t.reshape(VB // 4, 4, D)
    o_ref[...] = jnp.concatenate([t2[:, i, :] for i in range(4)], axis=1)


_pre = pl.pallas_call(
    _pre_body,
    grid=(PGRID,),
    in_specs=[pl.BlockSpec((D, VB), lambda i: (0, i))],
    out_specs=pl.BlockSpec((VB * D // IPG, IPG), lambda i: (i, 0)),
    out_shape=jax.ShapeDtypeStruct((QV * D // IPG, IPG), jnp.float32),
)


@functools.partial(
    pl.kernel,
    out_type=jax.ShapeDtypeStruct((TOTAL, D), jnp.float32),
    mesh=plsc.VectorSubcoreMesh(core_axis_name="c", subcore_axis_name="s"),
    scratch_types=[
        pltpu.VMEM((GPC, IPG), jnp.int32),
        pltpu.VMEM((GPC, IPG), jnp.int32),
        pltpu.VMEM((CHUNK, D), jnp.float32),
        pltpu.VMEM((CHUNK, D), jnp.float32),
        pltpu.SemaphoreType.DMA,
        pltpu.SemaphoreType.DMA,
        pltpu.SemaphoreType.DMA,
        pltpu.SemaphoreType.DMA,
    ],
    compiler_params=pltpu.CompilerParams(use_tc_tiling_on_sc=False),
)
def _gather_rows(idx_hbm, tab_hbm, out_hbm, idx_a, idx_b, rows_a, rows_b,
                 gsem_a, gsem_b, osem_a, osem_b):
    c = lax.axis_index("c")
    s_ax = lax.axis_index("s")
    wid = s_ax * NC + c
    r0_w = wid * PER_W

    def fire(g, idx_v, rows_v, gsem):
        r0 = r0_w + g * GPC
        pltpu.sync_copy(idx_hbm.at[pl.ds(r0, GPC)], idx_v)
        for j in range(GPC):
            pltpu.async_copy(
                tab_hbm.at[idx_v.at[j]],
                rows_v.at[pl.ds(j * IPG, IPG)],
                gsem,
            )
        return r0

    def drain_gathers(rows_v, gsem):
        for j in range(GPC):
            pltpu.make_async_copy(
                tab_hbm.at[idx_a.at[0]],
                rows_v.at[pl.ds(j * IPG, IPG)],
                gsem,
            ).wait()

    def store(r0, rows_v, osem):
        pltpu.async_copy(rows_v, out_hbm.at[pl.ds(r0 * IPG, CHUNK)], osem)

    def drain_store(rows_v, osem):
        pltpu.make_async_copy(
            rows_v, out_hbm.at[pl.ds(0, CHUNK)], osem
        ).wait()

    def pair_body(i, carry):
        @pl.when(i > 0)
        def _():
            drain_store(rows_a, osem_a)

        ra = fire(2 * i, idx_a, rows_a, gsem_a)

        @pl.when(i > 0)
        def _():
            drain_store(rows_b, osem_b)

        rb = fire(2 * i + 1, idx_b, rows_b, gsem_b)
        drain_gathers(rows_a, gsem_a)
        store(ra, rows_a, osem_a)
        drain_gathers(rows_b, gsem_b)
        store(rb, rows_b, osem_b)
        return carry

    lax.fori_loop(0, N_CHUNKS // 2, pair_body, 0)
    drain_store(rows_a, osem_a)
    drain_store(rows_b, osem_b)


def kernel(question, word2vec):
    idx = jnp.transpose(question).reshape(NBLK, IPG).astype(jnp.int32)
    tab = _pre(jnp.transpose(word2vec)).reshape(QV, D)
    out = _gather_rows(idx, tab)
    return out.reshape(S, B, D)


# speedup vs baseline: 2.4318x; 1.0016x over previous
import functools

import jax
import jax.numpy as jnp
from jax import lax
from jax.experimental import pallas as pl
from jax.experimental.pallas import tpu as pltpu
from jax.experimental.pallas import tpu_sc as plsc

B = 16384
S = 50
D = 32
QV = 1000000
TOTAL = B * S

NC = 2
NS = 16
NW = NC * NS

IPG = 128
NBLK = TOTAL // IPG
PER_W = NBLK // NW
GPC = 10
N_CHUNKS = PER_W // GPC
CHUNK = GPC * IPG

VB = 32768
PGRID = (QV + VB - 1) // VB


def _pre_body(x_ref, o_ref):
    t = jnp.tanh(x_ref[...]).T
    t2 = t.reshape(VB // 4, 4, D)
    o_ref[...] = jnp.concatenate([t2[:, i, :] for i in range(4)], axis=1)


_pre = pl.pallas_call(
    _pre_body,
    grid=(PGRID,),
    in_specs=[pl.BlockSpec((D, VB), lambda i: (0, i))],
    out_specs=pl.BlockSpec((VB * D // IPG, IPG), lambda i: (i, 0)),
    out_shape=jax.ShapeDtypeStruct((QV * D // IPG, IPG), jnp.float32),
)


@functools.partial(
    pl.kernel,
    out_type=jax.ShapeDtypeStruct((TOTAL, D), jnp.float32),
    mesh=plsc.VectorSubcoreMesh(core_axis_name="c", subcore_axis_name="s"),
    scratch_types=[
        pltpu.VMEM((GPC, IPG), jnp.int32),
        pltpu.VMEM((GPC, IPG), jnp.int32),
        pltpu.VMEM((CHUNK, D), jnp.float32),
        pltpu.VMEM((CHUNK, D), jnp.float32),
        pltpu.SemaphoreType.DMA,
        pltpu.SemaphoreType.DMA,
        pltpu.SemaphoreType.DMA,
        pltpu.SemaphoreType.DMA,
    ],
    compiler_params=pltpu.CompilerParams(use_tc_tiling_on_sc=False),
)
def _gather_rows(idx_hbm, tab_hbm, out_hbm, idx_a, idx_b, rows_a, rows_b,
                 gsem_a, gsem_b, osem_a, osem_b):
    c = lax.axis_index("c")
    s_ax = lax.axis_index("s")
    wid = s_ax * NC + c
    r0_w = wid * PER_W

    def fire(g, idx_v, rows_v, gsem):
        r0 = r0_w + g * GPC
        pltpu.sync_copy(idx_hbm.at[pl.ds(r0, GPC)], idx_v)
        for j in range(GPC):
            pltpu.async_copy(
                tab_hbm.at[idx_v.at[j]],
                rows_v.at[pl.ds(j * IPG, IPG)],
                gsem,
            )
        return r0

    def drain_gathers(rows_v, gsem):
        for j in range(GPC):
            pltpu.make_async_copy(
                tab_hbm.at[idx_a.at[0]],
                rows_v.at[pl.ds(j * IPG, IPG)],
                gsem,
            ).wait()

    def store(r0, rows_v, osem):
        pltpu.async_copy(rows_v, out_hbm.at[pl.ds(r0 * IPG, CHUNK)], osem)

    def drain_store(rows_v, osem):
        pltpu.make_async_copy(
            rows_v, out_hbm.at[pl.ds(0, CHUNK)], osem
        ).wait()

    def pair_body(i, carry):
        @pl.when(i > 0)
        def _():
            drain_store(rows_a, osem_a)

        ra = fire(2 * i, idx_a, rows_a, gsem_a)

        @pl.when(i > 0)
        def _():
            drain_store(rows_b, osem_b)

        rb = fire(2 * i + 1, idx_b, rows_b, gsem_b)
        drain_gathers(rows_a, gsem_a)
        store(ra, rows_a, osem_a)
        drain_gathers(rows_b, gsem_b)
        store(rb, rows_b, osem_b)
        return carry

    lax.fori_loop(0, N_CHUNKS // 2, pair_body, 0)
    drain_store(rows_a, osem_a)
    drain_store(rows_b, osem_b)


def kernel(question, word2vec):
    idx = jnp.transpose(question).reshape(NBLK, IPG).astype(jnp.int32)
    tab = _pre(jnp.transpose(word2vec)).reshape(QV, D)
    out = _gather_rows(idx, tab)
    return out.reshape(S, B, D)
